# baseline (speedup 1.0000x reference)
import functools

import jax
import jax.numpy as jnp
from jax import lax
from jax.experimental import pallas as pl
from jax.experimental.pallas import tpu as pltpu
from jax.experimental.pallas import tpu_sc as plsc

D_MODEL = 768
D_QUERY = 128
NUM_HEADS = 4
NUM_EXPERTS = 64
TOP_K = 2
NUM_SUB_KEYS = 8
EPS = 1e-5
N_TOKENS = 32768

NF = NUM_HEADS * D_QUERY
NS = NUM_HEADS * 2 * NUM_SUB_KEYS

NUM_WORKERS = 32
TOK_PER_W = N_TOKENS // NUM_WORKERS
GROUPS = TOK_PER_W // 16

BT1 = 1024
BT2 = 1024


def _stats_kernel(x_ref, w_ref, b_ref, sum_ref, sq_ref, q_ref):
    q = lax.dot_general(x_ref[...], w_ref[...], (((1,), (1,)), ((), ())),
                        preferred_element_type=jnp.float32)
    q = q + b_ref[...]
    q_ref[...] = q

    @pl.when(pl.program_id(0) == 0)
    def _():
        sum_ref[...] = jnp.zeros_like(sum_ref)
        sq_ref[...] = jnp.zeros_like(sq_ref)

    sum_ref[...] += jnp.sum(q, axis=0, keepdims=True)
    sq_ref[...] += jnp.sum(q * q, axis=0, keepdims=True)


def _stats(x, w512, b512):
    return pl.pallas_call(
        _stats_kernel,
        grid=(N_TOKENS // BT1,),
        in_specs=[
            pl.BlockSpec((BT1, D_MODEL), lambda i: (i, 0)),
            pl.BlockSpec((NF, D_MODEL), lambda i: (0, 0)),
            pl.BlockSpec((1, NF), lambda i: (0, 0)),
        ],
        out_specs=[
            pl.BlockSpec((1, NF), lambda i: (0, 0)),
            pl.BlockSpec((1, NF), lambda i: (0, 0)),
            pl.BlockSpec((BT1, NF), lambda i: (i, 0)),
        ],
        out_shape=[
            jax.ShapeDtypeStruct((1, NF), jnp.float32),
            jax.ShapeDtypeStruct((1, NF), jnp.float32),
            jax.ShapeDtypeStruct((N_TOKENS, NF), jnp.float32),
        ],
        compiler_params=pltpu.CompilerParams(
            dimension_semantics=("arbitrary",)),
    )(x, w512, b512)


def _scores_kernel(q_ref, mu_ref, ig_ref, be_ref, kbig_ref, out_ref):
    qn = (q_ref[...] - mu_ref[...]) * ig_ref[...] + be_ref[...]
    st = lax.dot_general(kbig_ref[...], qn, (((1,), (1,)), ((), ())),
                         preferred_element_type=jnp.float32)
    out_ref[...] = st[None]


def _scores(q, mu, ig, be512, kbig):
    full = lambda shape: pl.BlockSpec(shape, lambda i: (0,) * len(shape))
    return pl.pallas_call(
        _scores_kernel,
        grid=(N_TOKENS // BT2,),
        in_specs=[
            pl.BlockSpec((BT2, NF), lambda i: (i, 0)),
            full((1, NF)),
            full((1, NF)),
            full((1, NF)),
            full((NS, NF)),
        ],
        out_specs=pl.BlockSpec((1, NS, BT2), lambda i: (i, 0, 0)),
        out_shape=jax.ShapeDtypeStruct((NUM_WORKERS, NS, TOK_PER_W),
                                       jnp.float32),
        compiler_params=pltpu.CompilerParams(
            dimension_semantics=("arbitrary",)),
    )(q, mu, ig, be512, kbig)


def _top2_of_8(s_v, r, t0):
    v0 = s_v[r, pl.ds(t0, 16)]
    v1 = s_v[r + 1, pl.ds(t0, 16)]
    c0 = jnp.zeros((16,), jnp.int32)
    c1 = jnp.full((16,), 1, jnp.int32)
    gt = v1 > v0
    m1 = jnp.where(gt, v1, v0)
    i1 = jnp.where(gt, c1, c0)
    m2 = jnp.where(gt, v0, v1)
    i2 = jnp.where(gt, c0, c1)
    for i in range(2, NUM_SUB_KEYS):
        v = s_v[r + i, pl.ds(t0, 16)]
        ci = jnp.full((16,), i, jnp.int32)
        gt1 = v > m1
        gt2 = v > m2
        m2n = jnp.where(gt1, m1, jnp.where(gt2, v, m2))
        i2n = jnp.where(gt1, i1, jnp.where(gt2, ci, i2))
        m1 = jnp.where(gt1, v, m1)
        i1 = jnp.where(gt1, ci, i1)
        m2, i2 = m2n, i2n
    return m1, i1, m2, i2


def _select_body(s_hbm, outs_hbm, outi_hbm, s_v, os_v, oi_v):
    cid = lax.axis_index("c")
    sid = lax.axis_index("s")
    wid = sid * 2 + cid
    pltpu.sync_copy(s_hbm.at[wid], s_v)

    def group(g, carry):
        t0 = g * 16
        for h in range(NUM_HEADS):
            r = h * 2 * NUM_SUB_KEYS
            s11, i11, s12, i12 = _top2_of_8(s_v, r, t0)
            s21, i21, s22, i22 = _top2_of_8(s_v, r + NUM_SUB_KEYS, t0)
            best = s11 + s21
            alt1 = s11 + s22
            alt2 = s12 + s21
            use1 = alt1 >= alt2
            second = jnp.where(use1, alt1, alt2)
            f1 = jnp.where(use1, i11, i12)
            f2 = jnp.where(use1, i22, i21)
            e_best = i11 * NUM_SUB_KEYS + i21
            e_second = f1 * NUM_SUB_KEYS + f2
            e = jnp.exp(second - best)
            denom = e + 1.0
            p_best = 1.0 / denom
            p_second = e / denom
            os_v[h * 2, pl.ds(t0, 16)] = p_best
            os_v[h * 2 + 1, pl.ds(t0, 16)] = p_second
            oi_v[h * 2, pl.ds(t0, 16)] = e_best
            oi_v[h * 2 + 1, pl.ds(t0, 16)] = e_second
        return carry

    lax.fori_loop(0, GROUPS, group, 0)

    pltpu.sync_copy(os_v, outs_hbm.at[wid])
    pltpu.sync_copy(oi_v, outi_hbm.at[wid])


def _select(s_layout):
    mesh = plsc.VectorSubcoreMesh(core_axis_name="c", subcore_axis_name="s")
    hk = NUM_HEADS * TOP_K
    fn = functools.partial(
        pl.kernel,
        mesh=mesh,
        out_type=[
            jax.ShapeDtypeStruct((NUM_WORKERS, hk, TOK_PER_W), jnp.float32),
            jax.ShapeDtypeStruct((NUM_WORKERS, hk, TOK_PER_W), jnp.int32),
        ],
        scratch_types=[
            pltpu.VMEM((NS, TOK_PER_W), jnp.float32),
            pltpu.VMEM((hk, TOK_PER_W), jnp.float32),
            pltpu.VMEM((hk, TOK_PER_W), jnp.int32),
        ],
    )(_select_body)
    return fn(s_layout)


def kernel(x_flat, Wq, bq, gamma, beta, K1, K2):
    w512 = Wq.reshape(NF, D_MODEL)
    b512 = bq.reshape(1, NF)
    s_sum, s_sq, q = _stats(x_flat, w512, b512)

    n = jnp.float32(N_TOKENS)
    mu = s_sum / n
    var = s_sq / n - mu * mu
    ig = gamma.reshape(1, NF) / jnp.sqrt(var + EPS)

    half = D_QUERY // 2
    kbig = jnp.zeros((NS, NF), jnp.float32)
    for h in range(NUM_HEADS):
        r = h * 2 * NUM_SUB_KEYS
        c0 = h * D_QUERY
        kbig = kbig.at[r:r + NUM_SUB_KEYS, c0:c0 + half].set(K1)
        kbig = kbig.at[r + NUM_SUB_KEYS:r + 2 * NUM_SUB_KEYS,
                       c0 + half:c0 + D_QUERY].set(K2)

    s_layout = _scores(q, mu, ig, beta.reshape(1, NF), kbig)
    scores_t, idx_t = _select(s_layout)

    def _untranspose(a):
        return (a.reshape(NUM_WORKERS, NUM_HEADS, TOP_K, TOK_PER_W)
                 .transpose(0, 3, 1, 2)
                 .reshape(N_TOKENS, NUM_HEADS, TOP_K))

    return _untranspose(scores_t), _untranspose(idx_t)


# speedup vs baseline: 1.1598x; 1.1598x over previous
import functools

import jax
import jax.numpy as jnp
from jax import lax
from jax.experimental import pallas as pl
from jax.experimental.pallas import tpu as pltpu
from jax.experimental.pallas import tpu_sc as plsc

D_MODEL = 768
D_QUERY = 128
NUM_HEADS = 4
NUM_EXPERTS = 64
TOP_K = 2
NUM_SUB_KEYS = 8
EPS = 1e-5
N_TOKENS = 32768

NF = NUM_HEADS * D_QUERY
NS = NUM_HEADS * 2 * NUM_SUB_KEYS

NUM_WORKERS = 32
TOK_PER_W = N_TOKENS // NUM_WORKERS
GROUPS = TOK_PER_W // 16

BT1 = 2048
BT2 = 4096


def _stats_kernel(x_ref, w_ref, b_ref, sum_ref, sq_ref, q_ref):
    q = lax.dot_general(x_ref[...], w_ref[...], (((1,), (1,)), ((), ())),
                        preferred_element_type=jnp.float32)
    q = q + b_ref[...]
    q_ref[...] = q

    @pl.when(pl.program_id(0) == 0)
    def _():
        sum_ref[...] = jnp.zeros_like(sum_ref)
        sq_ref[...] = jnp.zeros_like(sq_ref)

    sum_ref[...] += jnp.sum(q, axis=0, keepdims=True)
    sq_ref[...] += jnp.sum(q * q, axis=0, keepdims=True)


def _stats(x, w512, b512):
    return pl.pallas_call(
        _stats_kernel,
        grid=(N_TOKENS // BT1,),
        in_specs=[
            pl.BlockSpec((BT1, D_MODEL), lambda i: (i, 0)),
            pl.BlockSpec((NF, D_MODEL), lambda i: (0, 0)),
            pl.BlockSpec((1, NF), lambda i: (0, 0)),
        ],
        out_specs=[
            pl.BlockSpec((1, NF), lambda i: (0, 0)),
            pl.BlockSpec((1, NF), lambda i: (0, 0)),
            pl.BlockSpec((BT1, NF), lambda i: (i, 0)),
        ],
        out_shape=[
            jax.ShapeDtypeStruct((1, NF), jnp.float32),
            jax.ShapeDtypeStruct((1, NF), jnp.float32),
            jax.ShapeDtypeStruct((N_TOKENS, NF), jnp.float32),
        ],
        compiler_params=pltpu.CompilerParams(
            dimension_semantics=("arbitrary",)),
    )(x, w512, b512)


def _scores_kernel(q_ref, mu_ref, ig_ref, be_ref, kbig_ref, out_ref):
    qn = (q_ref[...] - mu_ref[...]) * ig_ref[...] + be_ref[...]
    st = lax.dot_general(kbig_ref[...], qn, (((1,), (1,)), ((), ())),
                         preferred_element_type=jnp.float32)
    out_ref[...] = st


def _scores(q, mu, ig, be512, kbig):
    full = lambda shape: pl.BlockSpec(shape, lambda i: (0,) * len(shape))
    return pl.pallas_call(
        _scores_kernel,
        grid=(N_TOKENS // BT2,),
        in_specs=[
            pl.BlockSpec((BT2, NF), lambda i: (i, 0)),
            full((1, NF)),
            full((1, NF)),
            full((1, NF)),
            full((NS, NF)),
        ],
        out_specs=pl.BlockSpec((NS, BT2), lambda i: (0, i)),
        out_shape=jax.ShapeDtypeStruct((NS, N_TOKENS), jnp.float32),
        compiler_params=pltpu.CompilerParams(
            dimension_semantics=("arbitrary",)),
    )(q, mu, ig, be512, kbig)


def _top2_of_8(s_v, r, t0):
    v0 = s_v[r, pl.ds(t0, 16)]
    v1 = s_v[r + 1, pl.ds(t0, 16)]
    c0 = jnp.zeros((16,), jnp.int32)
    c1 = jnp.full((16,), 1, jnp.int32)
    gt = v1 > v0
    m1 = jnp.where(gt, v1, v0)
    i1 = jnp.where(gt, c1, c0)
    m2 = jnp.where(gt, v0, v1)
    i2 = jnp.where(gt, c0, c1)
    for i in range(2, NUM_SUB_KEYS):
        v = s_v[r + i, pl.ds(t0, 16)]
        ci = jnp.full((16,), i, jnp.int32)
        gt1 = v > m1
        gt2 = v > m2
        m2n = jnp.where(gt1, m1, jnp.where(gt2, v, m2))
        i2n = jnp.where(gt1, i1, jnp.where(gt2, ci, i2))
        m1 = jnp.where(gt1, v, m1)
        i1 = jnp.where(gt1, ci, i1)
        m2, i2 = m2n, i2n
    return m1, i1, m2, i2


def _select_body(s_hbm, outs_hbm, outi_hbm, s_v, os_v, oi_v):
    cid = lax.axis_index("c")
    sid = lax.axis_index("s")
    wid = sid * 2 + cid
    pltpu.sync_copy(s_hbm.at[:, pl.ds(wid * TOK_PER_W, TOK_PER_W)], s_v)

    def group(g, carry):
        t0 = g * 16
        for h in range(NUM_HEADS):
            r = h * 2 * NUM_SUB_KEYS
            s11, i11, s12, i12 = _top2_of_8(s_v, r, t0)
            s21, i21, s22, i22 = _top2_of_8(s_v, r + NUM_SUB_KEYS, t0)
            best = s11 + s21
            alt1 = s11 + s22
            alt2 = s12 + s21
            use1 = alt1 >= alt2
            second = jnp.where(use1, alt1, alt2)
            f1 = jnp.where(use1, i11, i12)
            f2 = jnp.where(use1, i22, i21)
            e_best = i11 * NUM_SUB_KEYS + i21
            e_second = f1 * NUM_SUB_KEYS + f2
            e = jnp.exp(second - best)
            denom = e + 1.0
            p_best = 1.0 / denom
            p_second = e / denom
            os_v[h * 2, pl.ds(t0, 16)] = p_best
            os_v[h * 2 + 1, pl.ds(t0, 16)] = p_second
            oi_v[h * 2, pl.ds(t0, 16)] = e_best
            oi_v[h * 2 + 1, pl.ds(t0, 16)] = e_second
        return carry

    lax.fori_loop(0, GROUPS, group, 0)

    pltpu.sync_copy(os_v, outs_hbm.at[wid])
    pltpu.sync_copy(oi_v, outi_hbm.at[wid])


def _select(s_layout):
    mesh = plsc.VectorSubcoreMesh(core_axis_name="c", subcore_axis_name="s")
    hk = NUM_HEADS * TOP_K
    fn = functools.partial(
        pl.kernel,
        mesh=mesh,
        out_type=[
            jax.ShapeDtypeStruct((NUM_WORKERS, hk, TOK_PER_W), jnp.float32),
            jax.ShapeDtypeStruct((NUM_WORKERS, hk, TOK_PER_W), jnp.int32),
        ],
        scratch_types=[
            pltpu.VMEM((NS, TOK_PER_W), jnp.float32),
            pltpu.VMEM((hk, TOK_PER_W), jnp.float32),
            pltpu.VMEM((hk, TOK_PER_W), jnp.int32),
        ],
    )(_select_body)
    return fn(s_layout)


def kernel(x_flat, Wq, bq, gamma, beta, K1, K2):
    w512 = Wq.reshape(NF, D_MODEL)
    b512 = bq.reshape(1, NF)
    s_sum, s_sq, q = _stats(x_flat, w512, b512)

    n = jnp.float32(N_TOKENS)
    mu = s_sum / n
    var = s_sq / n - mu * mu
    ig = gamma.reshape(1, NF) / jnp.sqrt(var + EPS)

    half = D_QUERY // 2
    kbig = jnp.zeros((NS, NF), jnp.float32)
    for h in range(NUM_HEADS):
        r = h * 2 * NUM_SUB_KEYS
        c0 = h * D_QUERY
        kbig = kbig.at[r:r + NUM_SUB_KEYS, c0:c0 + half].set(K1)
        kbig = kbig.at[r + NUM_SUB_KEYS:r + 2 * NUM_SUB_KEYS,
                       c0 + half:c0 + D_QUERY].set(K2)

    s_layout = _scores(q, mu, ig, beta.reshape(1, NF), kbig)
    scores_t, idx_t = _select(s_layout)

    def _untranspose(a):
        return (a.reshape(NUM_WORKERS, NUM_HEADS, TOP_K, TOK_PER_W)
                 .transpose(0, 3, 1, 2)
                 .reshape(N_TOKENS, NUM_HEADS, TOP_K))

    return _untranspose(scores_t), _untranspose(idx_t)


# speedup vs baseline: 1.1833x; 1.0203x over previous
import functools

import jax
import jax.numpy as jnp
from jax import lax
from jax.experimental import pallas as pl
from jax.experimental.pallas import tpu as pltpu
from jax.experimental.pallas import tpu_sc as plsc

D_MODEL = 768
D_QUERY = 128
NUM_HEADS = 4
NUM_EXPERTS = 64
TOP_K = 2
NUM_SUB_KEYS = 8
EPS = 1e-5
N_TOKENS = 32768

NF = NUM_HEADS * D_QUERY
NS = NUM_HEADS * 2 * NUM_SUB_KEYS

NUM_WORKERS = 32
TOK_PER_W = N_TOKENS // NUM_WORKERS
GROUPS = TOK_PER_W // 16

BT1 = 4096
BT2 = 4096


def _stats_kernel(x_ref, w_ref, b_ref, sum_ref, sq_ref):
    q = lax.dot_general(x_ref[...], w_ref[...], (((1,), (1,)), ((), ())),
                        preferred_element_type=jnp.float32)
    q = q + b_ref[...]

    @pl.when(pl.program_id(0) == 0)
    def _():
        sum_ref[...] = jnp.zeros_like(sum_ref)
        sq_ref[...] = jnp.zeros_like(sq_ref)

    sum_ref[...] += jnp.sum(q, axis=0, keepdims=True)
    sq_ref[...] += jnp.sum(q * q, axis=0, keepdims=True)


def _stats(x, w512, b512):
    return pl.pallas_call(
        _stats_kernel,
        grid=(N_TOKENS // BT1,),
        in_specs=[
            pl.BlockSpec((BT1, D_MODEL), lambda i: (i, 0)),
            pl.BlockSpec((NF, D_MODEL), lambda i: (0, 0)),
            pl.BlockSpec((1, NF), lambda i: (0, 0)),
        ],
        out_specs=[
            pl.BlockSpec((1, NF), lambda i: (0, 0)),
            pl.BlockSpec((1, NF), lambda i: (0, 0)),
        ],
        out_shape=[
            jax.ShapeDtypeStruct((1, NF), jnp.float32),
            jax.ShapeDtypeStruct((1, NF), jnp.float32),
        ],
        compiler_params=pltpu.CompilerParams(
            dimension_semantics=("arbitrary",)),
    )(x, w512, b512)


def _scores_kernel(x_ref, w_ref, b_ref, mu_ref, ig_ref, be_ref, kbig_ref,
                   out_ref):
    q = lax.dot_general(x_ref[...], w_ref[...], (((1,), (1,)), ((), ())),
                        preferred_element_type=jnp.float32)
    qn = (q + b_ref[...] - mu_ref[...]) * ig_ref[...] + be_ref[...]
    st = lax.dot_general(kbig_ref[...], qn, (((1,), (1,)), ((), ())),
                         preferred_element_type=jnp.float32)
    out_ref[...] = st


def _scores(x, w512, b512, mu, ig, be512, kbig):
    full = lambda shape: pl.BlockSpec(shape, lambda i: (0,) * len(shape))
    return pl.pallas_call(
        _scores_kernel,
        grid=(N_TOKENS // BT2,),
        in_specs=[
            pl.BlockSpec((BT2, D_MODEL), lambda i: (i, 0)),
            full((NF, D_MODEL)),
            full((1, NF)),
            full((1, NF)),
            full((1, NF)),
            full((1, NF)),
            full((NS, NF)),
        ],
        out_specs=pl.BlockSpec((NS, BT2), lambda i: (0, i)),
        out_shape=jax.ShapeDtypeStruct((NS, N_TOKENS), jnp.float32),
        compiler_params=pltpu.CompilerParams(
            dimension_semantics=("arbitrary",)),
    )(x, w512, b512, mu, ig, be512, kbig)


def _top2_of_8(s_v, r, t0):
    v0 = s_v[r, pl.ds(t0, 16)]
    v1 = s_v[r + 1, pl.ds(t0, 16)]
    c0 = jnp.zeros((16,), jnp.int32)
    c1 = jnp.full((16,), 1, jnp.int32)
    gt = v1 > v0
    m1 = jnp.where(gt, v1, v0)
    i1 = jnp.where(gt, c1, c0)
    m2 = jnp.where(gt, v0, v1)
    i2 = jnp.where(gt, c0, c1)
    for i in range(2, NUM_SUB_KEYS):
        v = s_v[r + i, pl.ds(t0, 16)]
        ci = jnp.full((16,), i, jnp.int32)
        gt1 = v > m1
        gt2 = v > m2
        m2n = jnp.where(gt1, m1, jnp.where(gt2, v, m2))
        i2n = jnp.where(gt1, i1, jnp.where(gt2, ci, i2))
        m1 = jnp.where(gt1, v, m1)
        i1 = jnp.where(gt1, ci, i1)
        m2, i2 = m2n, i2n
    return m1, i1, m2, i2


def _select_body(s_hbm, outs_hbm, outi_hbm, s_v, os_v, oi_v):
    cid = lax.axis_index("c")
    sid = lax.axis_index("s")
    wid = sid * 2 + cid
    pltpu.sync_copy(s_hbm.at[:, pl.ds(wid * TOK_PER_W, TOK_PER_W)], s_v)

    def group(g, carry):
        t0 = g * 16
        for h in range(NUM_HEADS):
            r = h * 2 * NUM_SUB_KEYS
            s11, i11, s12, i12 = _top2_of_8(s_v, r, t0)
            s21, i21, s22, i22 = _top2_of_8(s_v, r + NUM_SUB_KEYS, t0)
            best = s11 + s21
            alt1 = s11 + s22
            alt2 = s12 + s21
            use1 = alt1 >= alt2
            second = jnp.where(use1, alt1, alt2)
            f1 = jnp.where(use1, i11, i12)
            f2 = jnp.where(use1, i22, i21)
            e_best = i11 * NUM_SUB_KEYS + i21
            e_second = f1 * NUM_SUB_KEYS + f2
            e = jnp.exp(second - best)
            denom = e + 1.0
            p_best = 1.0 / denom
            p_second = e / denom
            os_v[h * 2, pl.ds(t0, 16)] = p_best
            os_v[h * 2 + 1, pl.ds(t0, 16)] = p_second
            oi_v[h * 2, pl.ds(t0, 16)] = e_best
            oi_v[h * 2 + 1, pl.ds(t0, 16)] = e_second
        return carry

    lax.fori_loop(0, GROUPS, group, 0)

    pltpu.sync_copy(os_v, outs_hbm.at[wid])
    pltpu.sync_copy(oi_v, outi_hbm.at[wid])


def _select(s_layout):
    mesh = plsc.VectorSubcoreMesh(core_axis_name="c", subcore_axis_name="s")
    hk = NUM_HEADS * TOP_K
    fn = functools.partial(
        pl.kernel,
        mesh=mesh,
        out_type=[
            jax.ShapeDtypeStruct((NUM_WORKERS, hk, TOK_PER_W), jnp.float32),
            jax.ShapeDtypeStruct((NUM_WORKERS, hk, TOK_PER_W), jnp.int32),
        ],
        scratch_types=[
            pltpu.VMEM((NS, TOK_PER_W), jnp.float32),
            pltpu.VMEM((hk, TOK_PER_W), jnp.float32),
            pltpu.VMEM((hk, TOK_PER_W), jnp.int32),
        ],
    )(_select_body)
    return fn(s_layout)


def kernel(x_flat, Wq, bq, gamma, beta, K1, K2):
    w512 = Wq.reshape(NF, D_MODEL)
    b512 = bq.reshape(1, NF)
    s_sum, s_sq = _stats(x_flat, w512, b512)

    n = jnp.float32(N_TOKENS)
    mu = s_sum / n
    var = s_sq / n - mu * mu
    ig = gamma.reshape(1, NF) / jnp.sqrt(var + EPS)

    half = D_QUERY // 2
    kbig = jnp.zeros((NS, NF), jnp.float32)
    for h in range(NUM_HEADS):
        r = h * 2 * NUM_SUB_KEYS
        c0 = h * D_QUERY
        kbig = kbig.at[r:r + NUM_SUB_KEYS, c0:c0 + half].set(K1)
        kbig = kbig.at[r + NUM_SUB_KEYS:r + 2 * NUM_SUB_KEYS,
                       c0 + half:c0 + D_QUERY].set(K2)

    s_layout = _scores(x_flat, w512, b512, mu, ig, beta.reshape(1, NF), kbig)
    scores_t, idx_t = _select(s_layout)

    def _untranspose(a):
        return (a.reshape(NUM_WORKERS, NUM_HEADS, TOP_K, TOK_PER_W)
                 .transpose(0, 3, 1, 2)
                 .reshape(N_TOKENS, NUM_HEADS, TOP_K))

    return _untranspose(scores_t), _untranspose(idx_t)
